# Initial kernel scaffold; baseline (speedup 1.0000x reference)
import jax
import jax.numpy as jnp
from jax.experimental import pallas as pl


def kernel(x, w, seg_i, idx_j, W_in2fac, W_fac2out, b_fac2out):
    raise NotImplementedError("write your pallas kernel here")


# speedup vs baseline: 2.7478x; 2.7478x over previous
import functools

import jax
import jax.numpy as jnp
from jax import lax
from jax.experimental import pallas as pl
from jax.experimental.pallas import tpu as pltpu
from jax.experimental.pallas import tpu_sc as plsc

N = 10000
E = 160000
NF = 256
FH = 128
NSUB = 16
EPS = E // NSUB
CHUNK = 80
NCHUNKS = EPS // CHUNK
ZR = 624
ZREM = N - ZR * NSUB
ZB = 156

_MXBLK = 1000


def _mm1_body(x_ref, w1_ref, o_ref):
    o_ref[0] = jnp.dot(x_ref[...], w1_ref[...],
                       preferred_element_type=jnp.float32)


def _mm2_body(c3_ref, w2_ref, b_ref, o_ref):
    a = jnp.dot(c3_ref[0], w2_ref[pl.ds(0, FH), :],
                preferred_element_type=jnp.float32)
    a += jnp.dot(c3_ref[1], w2_ref[pl.ds(FH, FH), :],
                 preferred_element_type=jnp.float32)
    a += b_ref[...]
    o_ref[...] = jax.nn.softplus(a)


def _sc_edge_body(f2, w, seg, idx2, conv2, acc, idx_v, seg_v, fj_v, wf_v,
                  zbuf, sem):
    c = lax.axis_index("c")
    s = lax.axis_index("s")

    zvec = jnp.zeros((16,), jnp.float32)

    def zfill(r, carry):
        for j in range(FH // 16):
            zbuf[r, pl.ds(j * 16, 16)] = zvec
        return carry

    lax.fori_loop(0, ZB, zfill, 0)
    for t in range(ZR // ZB):
        pltpu.sync_copy(zbuf, acc.at[pl.ds(s * ZR + t * ZB, ZB)])

    @pl.when(s == 0)
    def _zero_tail():
        pltpu.sync_copy(zbuf.at[pl.ds(0, ZREM)], acc.at[pl.ds(ZR * NSUB, ZREM)])

    plsc.subcore_barrier()

    def chunk_body(k, carry):
        base = s * EPS + k * CHUNK
        pltpu.sync_copy(idx2.at[pl.ds(c * E + base, CHUNK)], idx_v)
        pltpu.sync_copy(seg.at[pl.ds(base, CHUNK)], seg_v)
        gather = pltpu.make_async_copy(f2.at[idx_v], fj_v, sem)
        gather.start()
        pltpu.sync_copy(w.at[pl.ds(base, CHUNK), pl.ds(c * FH, FH)], wf_v)
        gather.wait()

        def mul_body(e, mcarry):
            for j in range(FH // 16):
                sl = pl.ds(j * 16, 16)
                wf_v[e, sl] = wf_v[e, sl] * fj_v[e, sl]
            return mcarry

        lax.fori_loop(0, CHUNK, mul_body, 0)
        pltpu.sync_copy(wf_v, acc.at[seg_v], add=True)
        return carry

    lax.fori_loop(0, NCHUNKS, chunk_body, 0)
    plsc.subcore_barrier()
    pltpu.sync_copy(acc.at[pl.ds(s * ZR, ZR)],
                    conv2.at[pl.ds(c * N + s * ZR, ZR)])

    @pl.when(s == 0)
    def _copy_tail():
        pltpu.sync_copy(acc.at[pl.ds(ZR * NSUB, ZREM)],
                        conv2.at[pl.ds(c * N + ZR * NSUB, ZREM)])


_sc_edge = pl.kernel(
    _sc_edge_body,
    out_type=jax.ShapeDtypeStruct((2 * N, FH), jnp.float32),
    name="sc_edge_cfconv",
    mesh=plsc.VectorSubcoreMesh(core_axis_name="c", subcore_axis_name="s",
                                num_cores=2, num_subcores=NSUB),
    scratch_types=[
        pltpu.VMEM_SHARED((N, FH), jnp.float32),
        pltpu.VMEM((CHUNK,), jnp.int32),
        pltpu.VMEM((CHUNK,), jnp.int32),
        pltpu.VMEM((CHUNK, FH), jnp.float32),
        pltpu.VMEM((CHUNK, FH), jnp.float32),
        pltpu.VMEM((ZB, FH), jnp.float32),
        pltpu.SemaphoreType.DMA,
    ],
)

_mm1 = pl.pallas_call(
    _mm1_body,
    grid=(2, N // _MXBLK),
    in_specs=[
        pl.BlockSpec((_MXBLK, NF), lambda h, i: (i, 0)),
        pl.BlockSpec((NF, FH), lambda h, i: (0, h)),
    ],
    out_specs=pl.BlockSpec((1, _MXBLK, FH), lambda h, i: (h, i, 0)),
    out_shape=jax.ShapeDtypeStruct((2, N, FH), jnp.float32),
)

_mm2 = pl.pallas_call(
    _mm2_body,
    grid=(N // _MXBLK,),
    in_specs=[
        pl.BlockSpec((2, _MXBLK, FH), lambda i: (0, i, 0)),
        pl.BlockSpec((NF, NF), lambda i: (0, 0)),
        pl.BlockSpec((1, NF), lambda i: (0, 0)),
    ],
    out_specs=pl.BlockSpec((_MXBLK, NF), lambda i: (i, 0)),
    out_shape=jax.ShapeDtypeStruct((N, NF), jnp.float32),
)


def kernel(x, w, seg_i, idx_j, W_in2fac, W_fac2out, b_fac2out):
    f3 = _mm1(x, W_in2fac)
    f2 = f3.reshape(2 * N, FH)
    idx2 = jnp.concatenate([idx_j, idx_j + N])
    conv2 = _sc_edge(f2, w, seg_i, idx2)
    c3 = conv2.reshape(2, N, FH)
    return _mm2(c3, W_fac2out, b_fac2out.reshape(1, NF))
